# baseline (speedup 1.0000x reference)
import functools

import jax
import jax.numpy as jnp
from jax import lax
from jax.experimental import pallas as pl
from jax.experimental.pallas import tpu as pltpu
from jax.experimental.pallas import tpu_sc as plsc

B = 16384
H = 64
V = 1000000
NW = 32
BPW = B // NW
ICHUNK = 128
NCHUNK = BPW // ICHUNK
L = 16

C = 512
NCH = 20
VPW = C * NCH
VS = NW * VPW
VT = V - VS
BK = 16384
assert VS % BK == 0
TC_GRID = (VT + BK - 1) // BK


def _tc_matvec_body(w8_ref, tu_ref, tv_ref, qu_ref, qv_ref):
    qu_ref[...] = jnp.dot(w8_ref[0:8, :], tu_ref[...],
                          preferred_element_type=jnp.float32)[0]
    qv_ref[...] = jnp.dot(w8_ref[8:16, :], tv_ref[...],
                          preferred_element_type=jnp.float32)[0]


_tc_matvec = pl.pallas_call(
    _tc_matvec_body,
    grid=(TC_GRID,),
    in_specs=[
        pl.BlockSpec((16, H), lambda i: (0, 0)),
        pl.BlockSpec((H, BK), lambda i: (0, i + VS // BK)),
        pl.BlockSpec((H, BK), lambda i: (0, i + VS // BK)),
    ],
    out_specs=[
        pl.BlockSpec((BK,), lambda i: (i,)),
        pl.BlockSpec((BK,), lambda i: (i,)),
    ],
    out_shape=[
        jax.ShapeDtypeStruct((VT,), jnp.float32),
        jax.ShapeDtypeStruct((VT,), jnp.float32),
    ],
    compiler_params=pltpu.CompilerParams(
        dimension_semantics=("arbitrary",)),
)


def _make_sc_sweep():
    mesh = plsc.VectorSubcoreMesh(core_axis_name="c", subcore_axis_name="s")

    @functools.partial(
        pl.kernel,
        mesh=mesh,
        out_type=[
            jax.ShapeDtypeStruct((VS,), jnp.float32),
            jax.ShapeDtypeStruct((VS,), jnp.float32),
        ],
        compiler_params=pltpu.CompilerParams(needs_layout_passes=False),
        scratch_types=[
            pltpu.VMEM((H, C), jnp.float32),
            pltpu.VMEM((H, C), jnp.float32),
            pltpu.VMEM((VPW,), jnp.float32),
            pltpu.VMEM((9 * L,), jnp.float32),
            pltpu.SemaphoreType.DMA,
            pltpu.SemaphoreType.DMA,
        ],
    )
    def k(wc_hbm, tu_hbm, tv_hbm, qu_hbm, qv_hbm,
          buf0, buf1, qout, wcv, sem0, sem1):
        wid = lax.axis_index("s") * 2 + lax.axis_index("c")
        vbase = wid * VPW
        pltpu.sync_copy(wc_hbm, wcv)
        bufs = (buf0, buf1)
        sems = (sem0, sem1)

        for t in range(2):
            src = tu_hbm if t == 0 else tv_hbm
            q_dst = qu_hbm if t == 0 else qv_hbm
            woff = t * H

            for b in range(2):
                pltpu.async_copy(
                    src.at[:, pl.ds(vbase + b * C, C)], bufs[b], sems[b])

            def chunk_pair(i, _, *, src=src, woff=woff):
                for b in range(2):
                    c_idx = 2 * i + b
                    pltpu.make_async_copy(
                        src.at[:, pl.ds(0, C)], bufs[b], sems[b]).wait()

                    def h_body(h, accs, *, b=b):
                        wh = plsc.load_gather(
                            wcv, [jnp.full((L,), woff + h, jnp.int32)])
                        return tuple(
                            accs[g] + wh * bufs[b][h, pl.ds(g * L, L)]
                            for g in range(C // L))

                    accs = lax.fori_loop(
                        0, H, h_body,
                        tuple(jnp.zeros((L,), jnp.float32)
                              for _ in range(C // L)))
                    for g in range(C // L):
                        qout[pl.ds(c_idx * C + g * L, L)] = accs[g]

                    @pl.when(c_idx + 2 < NCH)
                    def _(b=b, c_idx=c_idx, src=src):
                        pltpu.async_copy(
                            src.at[:, pl.ds(vbase + (c_idx + 2) * C, C)],
                            bufs[b], sems[b])
                return 0

            lax.fori_loop(0, NCH // 2, chunk_pair, 0)
            pltpu.sync_copy(qout, q_dst.at[pl.ds(vbase, VPW)])

    return k


def _make_sc_gather():
    mesh = plsc.VectorSubcoreMesh(core_axis_name="c", subcore_axis_name="s")

    @functools.partial(
        pl.kernel,
        mesh=mesh,
        out_type=jax.ShapeDtypeStruct((B,), jnp.float32),
        compiler_params=pltpu.CompilerParams(needs_layout_passes=False),
        scratch_types=[
            pltpu.VMEM((NCHUNK, ICHUNK), jnp.int32),
            pltpu.VMEM((NCHUNK, ICHUNK), jnp.int32),
            pltpu.VMEM((NCHUNK, ICHUNK), jnp.int32),
            pltpu.VMEM((NCHUNK, ICHUNK), jnp.int32),
            pltpu.VMEM((NCHUNK, ICHUNK), jnp.int32),
            pltpu.VMEM((NCHUNK, ICHUNK), jnp.int32),
            pltpu.VMEM((NCHUNK, ICHUNK), jnp.float32),
            pltpu.VMEM((NCHUNK, ICHUNK), jnp.float32),
            pltpu.VMEM((NCHUNK, ICHUNK), jnp.float32),
            pltpu.VMEM((NCHUNK, ICHUNK), jnp.float32),
            pltpu.VMEM((L,), jnp.float32),
            pltpu.VMEM((BPW,), jnp.float32),
            pltpu.SemaphoreType.DMA,
        ],
    )
    def k(u_hbm, v_hbm, cvec_hbm, qul_hbm, qvl_hbm, quh_hbm, qvh_hbm,
          out_hbm, idx_u, idx_v, ilo_u, ilo_v, ihi_u, ihi_v,
          vlo_u, vlo_v, vhi_u, vhi_v, cv, res, sem):
        wid = lax.axis_index("s") * 2 + lax.axis_index("c")
        base = wid * BPW

        pltpu.sync_copy(u_hbm.at[wid], idx_u)
        pltpu.sync_copy(v_hbm.at[wid], idx_v)
        pltpu.sync_copy(cvec_hbm, cv)

        for j in range(NCHUNK):
            for g in range(ICHUNK // L):
                sl = pl.ds(g * L, L)
                iu = idx_u[j, sl]
                iv = idx_v[j, sl]
                ilo_u[j, sl] = jnp.minimum(iu, VS - 1)
                ilo_v[j, sl] = jnp.minimum(iv, VS - 1)
                ihi_u[j, sl] = jnp.maximum(iu - VS, 0)
                ihi_v[j, sl] = jnp.maximum(iv - VS, 0)

        copies = []
        for j in range(NCHUNK):
            copies.append(pltpu.async_copy(
                qul_hbm.at[ilo_u.at[j]], vlo_u.at[j], sem))
            copies.append(pltpu.async_copy(
                qvl_hbm.at[ilo_v.at[j]], vlo_v.at[j], sem))
            copies.append(pltpu.async_copy(
                quh_hbm.at[ihi_u.at[j]], vhi_u.at[j], sem))
            copies.append(pltpu.async_copy(
                qvh_hbm.at[ihi_v.at[j]], vhi_v.at[j], sem))
        for cp in copies:
            cp.wait()

        cvec = cv[...]
        for j in range(NCHUNK):
            for g in range(ICHUNK // L):
                sl = pl.ds(g * L, L)
                mu = idx_u[j, sl] < VS
                mv = idx_v[j, sl] < VS
                a = jnp.where(mu, vlo_u[j, sl], vhi_u[j, sl])
                b = jnp.where(mv, vlo_v[j, sl], vhi_v[j, sl])
                z = 1.0 / (1.0 + jnp.exp(-(a + b + cvec)))
                res[pl.ds(j * ICHUNK + g * L, L)] = z

        pltpu.sync_copy(res, out_hbm.at[pl.ds(base, BPW)])

    return k


_sc_sweep = _make_sc_sweep()
_sc_gather = _make_sc_gather()


def kernel(u, v, table_u, table_v, W1, gamma, beta, W2):
    scale = (gamma * jax.lax.rsqrt(jnp.float32(1.0 + 1e-5))) * W2[0]
    weff = scale @ W1
    c = jnp.dot(beta, W2[0])
    w8 = jnp.zeros((16, H), jnp.float32)
    w8 = w8.at[0].set(weff[:H]).at[8].set(weff[H:])
    wc = jnp.concatenate([weff, jnp.full((L,), c, jnp.float32)])
    cvec = jnp.full((L,), c, jnp.float32)

    tuT = table_u.T
    tvT = table_v.T
    qu_lo, qv_lo = _sc_sweep(wc, tuT, tvT)
    qu_hi, qv_hi = _tc_matvec(w8, tuT, tvT)

    u3 = u.reshape(NW, NCHUNK, ICHUNK).astype(jnp.int32)
    v3 = v.reshape(NW, NCHUNK, ICHUNK).astype(jnp.int32)
    out = _sc_gather(u3, v3, cvec, qu_lo, qv_lo, qu_hi, qv_hi)
    return out.reshape(B, 1)


# speedup vs baseline: 1.2628x; 1.2628x over previous
import functools

import jax
import jax.numpy as jnp
from jax import lax
from jax.experimental import pallas as pl
from jax.experimental.pallas import tpu as pltpu
from jax.experimental.pallas import tpu_sc as plsc

B = 16384
H = 64
V = 1000000
NW = 32
BPW = B // NW
ICHUNK = 128
NCHUNK = BPW // ICHUNK
L = 16

C = 512
NCH = 20
VPW = C * NCH
VS = NW * VPW
VT = V - VS
BK = 16384
assert VS % BK == 0
TC_GRID = (VT + BK - 1) // BK


def _tc_matvec_body(w8_ref, tu_ref, tv_ref, qu_ref, qv_ref):
    qu_ref[...] = jnp.dot(w8_ref[0:8, :], tu_ref[...],
                          preferred_element_type=jnp.float32)[0]
    qv_ref[...] = jnp.dot(w8_ref[8:16, :], tv_ref[...],
                          preferred_element_type=jnp.float32)[0]


_tc_matvec = pl.pallas_call(
    _tc_matvec_body,
    grid=(TC_GRID,),
    in_specs=[
        pl.BlockSpec((16, H), lambda i: (0, 0)),
        pl.BlockSpec((H, BK), lambda i: (0, i + VS // BK)),
        pl.BlockSpec((H, BK), lambda i: (0, i + VS // BK)),
    ],
    out_specs=[
        pl.BlockSpec((BK,), lambda i: (i,)),
        pl.BlockSpec((BK,), lambda i: (i,)),
    ],
    out_shape=[
        jax.ShapeDtypeStruct((VT,), jnp.float32),
        jax.ShapeDtypeStruct((VT,), jnp.float32),
    ],
    compiler_params=pltpu.CompilerParams(
        dimension_semantics=("arbitrary",)),
)


def _make_sc_sweep():
    mesh = plsc.VectorSubcoreMesh(core_axis_name="c", subcore_axis_name="s")

    @functools.partial(
        pl.kernel,
        mesh=mesh,
        out_type=[
            jax.ShapeDtypeStruct((VS,), jnp.float32),
            jax.ShapeDtypeStruct((VS,), jnp.float32),
        ],
        compiler_params=pltpu.CompilerParams(needs_layout_passes=False),
        scratch_types=[
            pltpu.VMEM((H, C), jnp.float32),
            pltpu.VMEM((H, C), jnp.float32),
            pltpu.VMEM((VPW,), jnp.float32),
            pltpu.VMEM((9 * L,), jnp.float32),
            pltpu.SemaphoreType.DMA,
            pltpu.SemaphoreType.DMA,
        ],
    )
    def k(wc_hbm, tu_hbm, tv_hbm, qu_hbm, qv_hbm,
          buf0, buf1, qout, wcv, sem0, sem1):
        wid = lax.axis_index("s") * 2 + lax.axis_index("c")
        vbase = wid * VPW
        pltpu.sync_copy(wc_hbm, wcv)
        bufs = (buf0, buf1)
        sems = (sem0, sem1)

        for t in range(2):
            src = tu_hbm if t == 0 else tv_hbm
            q_dst = qu_hbm if t == 0 else qv_hbm
            woff = t * H

            for b in range(2):
                pltpu.async_copy(
                    src.at[:, pl.ds(vbase + b * C, C)], bufs[b], sems[b])

            def chunk_pair(i, _, *, src=src, woff=woff):
                for b in range(2):
                    c_idx = 2 * i + b
                    pltpu.make_async_copy(
                        src.at[:, pl.ds(0, C)], bufs[b], sems[b]).wait()

                    def h_body(h, accs, *, b=b):
                        wh = plsc.load_gather(
                            wcv, [jnp.full((L,), woff + h, jnp.int32)])
                        return tuple(
                            accs[g] + wh * bufs[b][h, pl.ds(g * L, L)]
                            for g in range(C // L))

                    accs = lax.fori_loop(
                        0, H, h_body,
                        tuple(jnp.zeros((L,), jnp.float32)
                              for _ in range(C // L)))
                    for g in range(C // L):
                        qout[pl.ds(c_idx * C + g * L, L)] = accs[g]

                    @pl.when(c_idx + 2 < NCH)
                    def _(b=b, c_idx=c_idx, src=src):
                        pltpu.async_copy(
                            src.at[:, pl.ds(vbase + (c_idx + 2) * C, C)],
                            bufs[b], sems[b])
                return 0

            lax.fori_loop(0, NCH // 2, chunk_pair, 0)
            pltpu.sync_copy(qout, q_dst.at[pl.ds(vbase, VPW)])

    return k


def _make_sc_gather():
    mesh = plsc.VectorSubcoreMesh(core_axis_name="c", subcore_axis_name="s")

    @functools.partial(
        pl.kernel,
        mesh=mesh,
        out_type=jax.ShapeDtypeStruct((B,), jnp.float32),
        compiler_params=pltpu.CompilerParams(needs_layout_passes=False),
        scratch_types=[
            pltpu.VMEM((NCHUNK, ICHUNK), jnp.int32),
            pltpu.VMEM((NCHUNK, ICHUNK), jnp.int32),
            pltpu.VMEM((NCHUNK, ICHUNK), jnp.int32),
            pltpu.VMEM((NCHUNK, ICHUNK), jnp.int32),
            pltpu.VMEM((NCHUNK, ICHUNK), jnp.int32),
            pltpu.VMEM((NCHUNK, ICHUNK), jnp.int32),
            pltpu.VMEM((NCHUNK, ICHUNK), jnp.float32),
            pltpu.VMEM((NCHUNK, ICHUNK), jnp.float32),
            pltpu.VMEM((NCHUNK, ICHUNK), jnp.float32),
            pltpu.VMEM((NCHUNK, ICHUNK), jnp.float32),
            pltpu.VMEM((L,), jnp.float32),
            pltpu.VMEM((BPW,), jnp.float32),
            pltpu.SemaphoreType.DMA,
        ],
    )
    def k(u_hbm, v_hbm, cvec_hbm, qul_hbm, qvl_hbm, quh_hbm, qvh_hbm,
          out_hbm, idx_u, idx_v, ilo_u, ilo_v, ihi_u, ihi_v,
          vlo_u, vlo_v, vhi_u, vhi_v, cv, res, sem):
        wid = lax.axis_index("s") * 2 + lax.axis_index("c")
        base = wid * BPW

        pltpu.sync_copy(u_hbm.at[wid], idx_u)
        pltpu.sync_copy(v_hbm.at[wid], idx_v)
        pltpu.sync_copy(cvec_hbm, cv)

        for j in range(NCHUNK):
            for g in range(ICHUNK // L):
                sl = pl.ds(g * L, L)
                iu = idx_u[j, sl]
                iv = idx_v[j, sl]
                fu = iu & 4095
                fv = iv & 4095
                ilo_u[j, sl] = jnp.where(iu < VS, iu, fu)
                ilo_v[j, sl] = jnp.where(iv < VS, iv, fv)
                ihi_u[j, sl] = jnp.where(iu >= VS, iu - VS, fu)
                ihi_v[j, sl] = jnp.where(iv >= VS, iv - VS, fv)

        copies = []
        for j in range(NCHUNK):
            copies.append(pltpu.async_copy(
                qul_hbm.at[ilo_u.at[j]], vlo_u.at[j], sem))
            copies.append(pltpu.async_copy(
                qvl_hbm.at[ilo_v.at[j]], vlo_v.at[j], sem))
            copies.append(pltpu.async_copy(
                quh_hbm.at[ihi_u.at[j]], vhi_u.at[j], sem))
            copies.append(pltpu.async_copy(
                qvh_hbm.at[ihi_v.at[j]], vhi_v.at[j], sem))
        for cp in copies:
            cp.wait()

        cvec = cv[...]
        for j in range(NCHUNK):
            for g in range(ICHUNK // L):
                sl = pl.ds(g * L, L)
                mu = idx_u[j, sl] < VS
                mv = idx_v[j, sl] < VS
                a = jnp.where(mu, vlo_u[j, sl], vhi_u[j, sl])
                b = jnp.where(mv, vlo_v[j, sl], vhi_v[j, sl])
                z = 1.0 / (1.0 + jnp.exp(-(a + b + cvec)))
                res[pl.ds(j * ICHUNK + g * L, L)] = z

        pltpu.sync_copy(res, out_hbm.at[pl.ds(base, BPW)])

    return k


_sc_sweep = _make_sc_sweep()
_sc_gather = _make_sc_gather()


def kernel(u, v, table_u, table_v, W1, gamma, beta, W2):
    scale = (gamma * jax.lax.rsqrt(jnp.float32(1.0 + 1e-5))) * W2[0]
    weff = scale @ W1
    c = jnp.dot(beta, W2[0])
    w8 = jnp.zeros((16, H), jnp.float32)
    w8 = w8.at[0].set(weff[:H]).at[8].set(weff[H:])
    wc = jnp.concatenate([weff, jnp.full((L,), c, jnp.float32)])
    cvec = jnp.full((L,), c, jnp.float32)

    tuT = table_u.T
    tvT = table_v.T
    qu_lo, qv_lo = _sc_sweep(wc, tuT, tvT)
    qu_hi, qv_hi = _tc_matvec(w8, tuT, tvT)

    u3 = u.reshape(NW, NCHUNK, ICHUNK).astype(jnp.int32)
    v3 = v.reshape(NW, NCHUNK, ICHUNK).astype(jnp.int32)
    out = _sc_gather(u3, v3, cvec, qu_lo, qv_lo, qu_hi, qv_hi)
    return out.reshape(B, 1)


# speedup vs baseline: 1.2716x; 1.0070x over previous
import functools

import jax
import jax.numpy as jnp
from jax import lax
from jax.experimental import pallas as pl
from jax.experimental.pallas import tpu as pltpu
from jax.experimental.pallas import tpu_sc as plsc

B = 16384
H = 64
V = 1000000
NW = 32
BPW = B // NW
ICHUNK = 128
NCHUNK = BPW // ICHUNK
L = 16

C = 512
NCH = 14
VPW = C * NCH
VS = NW * VPW
VT = V - VS
BK = 16384
assert VS % BK == 0
TC_GRID = (VT + BK - 1) // BK


def _tc_matvec_body(w8_ref, tu_ref, tv_ref, qu_ref, qv_ref):
    qu_ref[...] = jnp.dot(w8_ref[0:8, :], tu_ref[...],
                          preferred_element_type=jnp.float32)[0]
    qv_ref[...] = jnp.dot(w8_ref[8:16, :], tv_ref[...],
                          preferred_element_type=jnp.float32)[0]


_tc_matvec = pl.pallas_call(
    _tc_matvec_body,
    grid=(TC_GRID,),
    in_specs=[
        pl.BlockSpec((16, H), lambda i: (0, 0)),
        pl.BlockSpec((H, BK), lambda i: (0, i + VS // BK)),
        pl.BlockSpec((H, BK), lambda i: (0, i + VS // BK)),
    ],
    out_specs=[
        pl.BlockSpec((BK,), lambda i: (i,)),
        pl.BlockSpec((BK,), lambda i: (i,)),
    ],
    out_shape=[
        jax.ShapeDtypeStruct((VT,), jnp.float32),
        jax.ShapeDtypeStruct((VT,), jnp.float32),
    ],
    compiler_params=pltpu.CompilerParams(
        dimension_semantics=("arbitrary",)),
)


def _make_sc_sweep():
    mesh = plsc.VectorSubcoreMesh(core_axis_name="c", subcore_axis_name="s")

    @functools.partial(
        pl.kernel,
        mesh=mesh,
        out_type=[
            jax.ShapeDtypeStruct((VS,), jnp.float32),
            jax.ShapeDtypeStruct((VS,), jnp.float32),
        ],
        compiler_params=pltpu.CompilerParams(needs_layout_passes=False),
        scratch_types=[
            pltpu.VMEM((H, C), jnp.float32),
            pltpu.VMEM((H, C), jnp.float32),
            pltpu.VMEM((VPW,), jnp.float32),
            pltpu.VMEM((9 * L,), jnp.float32),
            pltpu.SemaphoreType.DMA,
            pltpu.SemaphoreType.DMA,
        ],
    )
    def k(wc_hbm, tu_hbm, tv_hbm, qu_hbm, qv_hbm,
          buf0, buf1, qout, wcv, sem0, sem1):
        wid = lax.axis_index("s") * 2 + lax.axis_index("c")
        vbase = wid * VPW
        pltpu.sync_copy(wc_hbm, wcv)
        bufs = (buf0, buf1)
        sems = (sem0, sem1)

        for t in range(2):
            src = tu_hbm if t == 0 else tv_hbm
            q_dst = qu_hbm if t == 0 else qv_hbm
            woff = t * H

            for b in range(2):
                pltpu.async_copy(
                    src.at[:, pl.ds(vbase + b * C, C)], bufs[b], sems[b])

            def chunk_pair(i, _, *, src=src, woff=woff):
                for b in range(2):
                    c_idx = 2 * i + b
                    pltpu.make_async_copy(
                        src.at[:, pl.ds(0, C)], bufs[b], sems[b]).wait()

                    def h_body(h, accs, *, b=b):
                        wh = plsc.load_gather(
                            wcv, [jnp.full((L,), woff + h, jnp.int32)])
                        return tuple(
                            accs[g] + wh * bufs[b][h, pl.ds(g * L, L)]
                            for g in range(C // L))

                    accs = lax.fori_loop(
                        0, H, h_body,
                        tuple(jnp.zeros((L,), jnp.float32)
                              for _ in range(C // L)))
                    for g in range(C // L):
                        qout[pl.ds(c_idx * C + g * L, L)] = accs[g]

                    @pl.when(c_idx + 2 < NCH)
                    def _(b=b, c_idx=c_idx, src=src):
                        pltpu.async_copy(
                            src.at[:, pl.ds(vbase + (c_idx + 2) * C, C)],
                            bufs[b], sems[b])
                return 0

            lax.fori_loop(0, NCH // 2, chunk_pair, 0)
            pltpu.sync_copy(qout, q_dst.at[pl.ds(vbase, VPW)])

    return k


def _make_sc_gather():
    mesh = plsc.VectorSubcoreMesh(core_axis_name="c", subcore_axis_name="s")

    @functools.partial(
        pl.kernel,
        mesh=mesh,
        out_type=jax.ShapeDtypeStruct((B,), jnp.float32),
        compiler_params=pltpu.CompilerParams(needs_layout_passes=False),
        scratch_types=[
            pltpu.VMEM((NCHUNK, ICHUNK), jnp.int32),
            pltpu.VMEM((NCHUNK, ICHUNK), jnp.int32),
            pltpu.VMEM((NCHUNK, ICHUNK), jnp.int32),
            pltpu.VMEM((NCHUNK, ICHUNK), jnp.int32),
            pltpu.VMEM((NCHUNK, ICHUNK), jnp.int32),
            pltpu.VMEM((NCHUNK, ICHUNK), jnp.int32),
            pltpu.VMEM((NCHUNK, ICHUNK), jnp.float32),
            pltpu.VMEM((NCHUNK, ICHUNK), jnp.float32),
            pltpu.VMEM((NCHUNK, ICHUNK), jnp.float32),
            pltpu.VMEM((NCHUNK, ICHUNK), jnp.float32),
            pltpu.VMEM((L,), jnp.float32),
            pltpu.VMEM((BPW,), jnp.float32),
            pltpu.SemaphoreType.DMA,
        ],
    )
    def k(u_hbm, v_hbm, cvec_hbm, qul_hbm, qvl_hbm, quh_hbm, qvh_hbm,
          out_hbm, idx_u, idx_v, ilo_u, ilo_v, ihi_u, ihi_v,
          vlo_u, vlo_v, vhi_u, vhi_v, cv, res, sem):
        wid = lax.axis_index("s") * 2 + lax.axis_index("c")
        base = wid * BPW

        pltpu.sync_copy(u_hbm.at[wid], idx_u)
        pltpu.sync_copy(v_hbm.at[wid], idx_v)
        pltpu.sync_copy(cvec_hbm, cv)

        for j in range(NCHUNK):
            for g in range(ICHUNK // L):
                sl = pl.ds(g * L, L)
                iu = idx_u[j, sl]
                iv = idx_v[j, sl]
                fu = iu & 4095
                fv = iv & 4095
                ilo_u[j, sl] = jnp.where(iu < VS, iu, fu)
                ilo_v[j, sl] = jnp.where(iv < VS, iv, fv)
                ihi_u[j, sl] = jnp.where(iu >= VS, iu - VS, fu)
                ihi_v[j, sl] = jnp.where(iv >= VS, iv - VS, fv)

        copies = []
        for j in range(NCHUNK):
            copies.append(pltpu.async_copy(
                qul_hbm.at[ilo_u.at[j]], vlo_u.at[j], sem))
            copies.append(pltpu.async_copy(
                qvl_hbm.at[ilo_v.at[j]], vlo_v.at[j], sem))
            copies.append(pltpu.async_copy(
                quh_hbm.at[ihi_u.at[j]], vhi_u.at[j], sem))
            copies.append(pltpu.async_copy(
                qvh_hbm.at[ihi_v.at[j]], vhi_v.at[j], sem))
        for cp in copies:
            cp.wait()

        cvec = cv[...]
        for j in range(NCHUNK):
            for g in range(ICHUNK // L):
                sl = pl.ds(g * L, L)
                mu = idx_u[j, sl] < VS
                mv = idx_v[j, sl] < VS
                a = jnp.where(mu, vlo_u[j, sl], vhi_u[j, sl])
                b = jnp.where(mv, vlo_v[j, sl], vhi_v[j, sl])
                z = 1.0 / (1.0 + jnp.exp(-(a + b + cvec)))
                res[pl.ds(j * ICHUNK + g * L, L)] = z

        pltpu.sync_copy(res, out_hbm.at[pl.ds(base, BPW)])

    return k


_sc_sweep = _make_sc_sweep()
_sc_gather = _make_sc_gather()


def kernel(u, v, table_u, table_v, W1, gamma, beta, W2):
    scale = (gamma * jax.lax.rsqrt(jnp.float32(1.0 + 1e-5))) * W2[0]
    weff = scale @ W1
    c = jnp.dot(beta, W2[0])
    w8 = jnp.zeros((16, H), jnp.float32)
    w8 = w8.at[0].set(weff[:H]).at[8].set(weff[H:])
    wc = jnp.concatenate([weff, jnp.full((L,), c, jnp.float32)])
    cvec = jnp.full((L,), c, jnp.float32)

    tuT = table_u.T
    tvT = table_v.T
    qu_lo, qv_lo = _sc_sweep(wc, tuT, tvT)
    qu_hi, qv_hi = _tc_matvec(w8, tuT, tvT)

    u3 = u.reshape(NW, NCHUNK, ICHUNK).astype(jnp.int32)
    v3 = v.reshape(NW, NCHUNK, ICHUNK).astype(jnp.int32)
    out = _sc_gather(u3, v3, cvec, qu_lo, qv_lo, qu_hi, qv_hi)
    return out.reshape(B, 1)


# speedup vs baseline: 1.2763x; 1.0037x over previous
import functools

import jax
import jax.numpy as jnp
from jax import lax
from jax.experimental import pallas as pl
from jax.experimental.pallas import tpu as pltpu
from jax.experimental.pallas import tpu_sc as plsc

B = 16384
H = 64
V = 1000000
NW = 32
BPW = B // NW
ICHUNK = 128
NCHUNK = BPW // ICHUNK
L = 16

C = 512
NCH = 4
VPW = C * NCH
VS = NW * VPW
VT = V - VS
BK = 16384
assert VS % BK == 0
TC_GRID = (VT + BK - 1) // BK


def _tc_matvec_body(w8_ref, tu_ref, tv_ref, qu_ref, qv_ref):
    qu_ref[...] = jnp.dot(w8_ref[0:8, :], tu_ref[...],
                          preferred_element_type=jnp.float32)[0]
    qv_ref[...] = jnp.dot(w8_ref[8:16, :], tv_ref[...],
                          preferred_element_type=jnp.float32)[0]


_tc_matvec = pl.pallas_call(
    _tc_matvec_body,
    grid=(TC_GRID,),
    in_specs=[
        pl.BlockSpec((16, H), lambda i: (0, 0)),
        pl.BlockSpec((H, BK), lambda i: (0, i + VS // BK)),
        pl.BlockSpec((H, BK), lambda i: (0, i + VS // BK)),
    ],
    out_specs=[
        pl.BlockSpec((BK,), lambda i: (i,)),
        pl.BlockSpec((BK,), lambda i: (i,)),
    ],
    out_shape=[
        jax.ShapeDtypeStruct((VT,), jnp.float32),
        jax.ShapeDtypeStruct((VT,), jnp.float32),
    ],
    compiler_params=pltpu.CompilerParams(
        dimension_semantics=("arbitrary",)),
)


def _make_sc_sweep():
    mesh = plsc.VectorSubcoreMesh(core_axis_name="c", subcore_axis_name="s")

    @functools.partial(
        pl.kernel,
        mesh=mesh,
        out_type=[
            jax.ShapeDtypeStruct((VS,), jnp.float32),
            jax.ShapeDtypeStruct((VS,), jnp.float32),
        ],
        compiler_params=pltpu.CompilerParams(needs_layout_passes=False),
        scratch_types=[
            pltpu.VMEM((H, C), jnp.float32),
            pltpu.VMEM((H, C), jnp.float32),
            pltpu.VMEM((VPW,), jnp.float32),
            pltpu.VMEM((9 * L,), jnp.float32),
            pltpu.SemaphoreType.DMA,
            pltpu.SemaphoreType.DMA,
        ],
    )
    def k(wc_hbm, tu_hbm, tv_hbm, qu_hbm, qv_hbm,
          buf0, buf1, qout, wcv, sem0, sem1):
        wid = lax.axis_index("s") * 2 + lax.axis_index("c")
        vbase = wid * VPW
        pltpu.sync_copy(wc_hbm, wcv)
        bufs = (buf0, buf1)
        sems = (sem0, sem1)

        for t in range(2):
            src = tu_hbm if t == 0 else tv_hbm
            q_dst = qu_hbm if t == 0 else qv_hbm
            woff = t * H

            for b in range(2):
                pltpu.async_copy(
                    src.at[:, pl.ds(vbase + b * C, C)], bufs[b], sems[b])

            def chunk_pair(i, _, *, src=src, woff=woff):
                for b in range(2):
                    c_idx = 2 * i + b
                    pltpu.make_async_copy(
                        src.at[:, pl.ds(0, C)], bufs[b], sems[b]).wait()

                    def h_body(h, accs, *, b=b):
                        wh = plsc.load_gather(
                            wcv, [jnp.full((L,), woff + h, jnp.int32)])
                        return tuple(
                            accs[g] + wh * bufs[b][h, pl.ds(g * L, L)]
                            for g in range(C // L))

                    accs = lax.fori_loop(
                        0, H, h_body,
                        tuple(jnp.zeros((L,), jnp.float32)
                              for _ in range(C // L)))
                    for g in range(C // L):
                        qout[pl.ds(c_idx * C + g * L, L)] = accs[g]

                    @pl.when(c_idx + 2 < NCH)
                    def _(b=b, c_idx=c_idx, src=src):
                        pltpu.async_copy(
                            src.at[:, pl.ds(vbase + (c_idx + 2) * C, C)],
                            bufs[b], sems[b])
                return 0

            lax.fori_loop(0, NCH // 2, chunk_pair, 0)
            pltpu.sync_copy(qout, q_dst.at[pl.ds(vbase, VPW)])

    return k


def _make_sc_gather():
    mesh = plsc.VectorSubcoreMesh(core_axis_name="c", subcore_axis_name="s")

    @functools.partial(
        pl.kernel,
        mesh=mesh,
        out_type=jax.ShapeDtypeStruct((B,), jnp.float32),
        compiler_params=pltpu.CompilerParams(needs_layout_passes=False),
        scratch_types=[
            pltpu.VMEM((NCHUNK, ICHUNK), jnp.int32),
            pltpu.VMEM((NCHUNK, ICHUNK), jnp.int32),
            pltpu.VMEM((NCHUNK, ICHUNK), jnp.int32),
            pltpu.VMEM((NCHUNK, ICHUNK), jnp.int32),
            pltpu.VMEM((NCHUNK, ICHUNK), jnp.int32),
            pltpu.VMEM((NCHUNK, ICHUNK), jnp.int32),
            pltpu.VMEM((NCHUNK, ICHUNK), jnp.float32),
            pltpu.VMEM((NCHUNK, ICHUNK), jnp.float32),
            pltpu.VMEM((NCHUNK, ICHUNK), jnp.float32),
            pltpu.VMEM((NCHUNK, ICHUNK), jnp.float32),
            pltpu.VMEM((L,), jnp.float32),
            pltpu.VMEM((BPW,), jnp.float32),
            pltpu.SemaphoreType.DMA,
        ],
    )
    def k(u_hbm, v_hbm, cvec_hbm, qul_hbm, qvl_hbm, quh_hbm, qvh_hbm,
          out_hbm, idx_u, idx_v, ilo_u, ilo_v, ihi_u, ihi_v,
          vlo_u, vlo_v, vhi_u, vhi_v, cv, res, sem):
        wid = lax.axis_index("s") * 2 + lax.axis_index("c")
        base = wid * BPW

        pltpu.sync_copy(u_hbm.at[wid], idx_u)
        pltpu.sync_copy(v_hbm.at[wid], idx_v)
        pltpu.sync_copy(cvec_hbm, cv)

        for j in range(NCHUNK):
            for g in range(ICHUNK // L):
                sl = pl.ds(g * L, L)
                iu = idx_u[j, sl]
                iv = idx_v[j, sl]
                fu = iu & 4095
                fv = iv & 4095
                ilo_u[j, sl] = jnp.where(iu < VS, iu, fu)
                ilo_v[j, sl] = jnp.where(iv < VS, iv, fv)
                ihi_u[j, sl] = jnp.where(iu >= VS, iu - VS, fu)
                ihi_v[j, sl] = jnp.where(iv >= VS, iv - VS, fv)

        copies = []
        for j in range(NCHUNK):
            copies.append(pltpu.async_copy(
                qul_hbm.at[ilo_u.at[j]], vlo_u.at[j], sem))
            copies.append(pltpu.async_copy(
                qvl_hbm.at[ilo_v.at[j]], vlo_v.at[j], sem))
            copies.append(pltpu.async_copy(
                quh_hbm.at[ihi_u.at[j]], vhi_u.at[j], sem))
            copies.append(pltpu.async_copy(
                qvh_hbm.at[ihi_v.at[j]], vhi_v.at[j], sem))
        for cp in copies:
            cp.wait()

        cvec = cv[...]
        for j in range(NCHUNK):
            for g in range(ICHUNK // L):
                sl = pl.ds(g * L, L)
                mu = idx_u[j, sl] < VS
                mv = idx_v[j, sl] < VS
                a = jnp.where(mu, vlo_u[j, sl], vhi_u[j, sl])
                b = jnp.where(mv, vlo_v[j, sl], vhi_v[j, sl])
                z = 1.0 / (1.0 + jnp.exp(-(a + b + cvec)))
                res[pl.ds(j * ICHUNK + g * L, L)] = z

        pltpu.sync_copy(res, out_hbm.at[pl.ds(base, BPW)])

    return k


_sc_sweep = _make_sc_sweep()
_sc_gather = _make_sc_gather()


def kernel(u, v, table_u, table_v, W1, gamma, beta, W2):
    scale = (gamma * jax.lax.rsqrt(jnp.float32(1.0 + 1e-5))) * W2[0]
    weff = scale @ W1
    c = jnp.dot(beta, W2[0])
    w8 = jnp.zeros((16, H), jnp.float32)
    w8 = w8.at[0].set(weff[:H]).at[8].set(weff[H:])
    wc = jnp.concatenate([weff, jnp.full((L,), c, jnp.float32)])
    cvec = jnp.full((L,), c, jnp.float32)

    tuT = table_u.T
    tvT = table_v.T
    qu_lo, qv_lo = _sc_sweep(wc, tuT, tvT)
    qu_hi, qv_hi = _tc_matvec(w8, tuT, tvT)

    u3 = u.reshape(NW, NCHUNK, ICHUNK).astype(jnp.int32)
    v3 = v.reshape(NW, NCHUNK, ICHUNK).astype(jnp.int32)
    out = _sc_gather(u3, v3, cvec, qu_lo, qv_lo, qu_hi, qv_hi)
    return out.reshape(B, 1)


# speedup vs baseline: 1.3395x; 1.0495x over previous
import functools

import jax
import jax.numpy as jnp
from jax import lax
from jax.experimental import pallas as pl
from jax.experimental.pallas import tpu as pltpu
from jax.experimental.pallas import tpu_sc as plsc

B = 16384
H = 64
V = 1000000
NW = 32
BPW = B // NW
ICHUNK = 128
NCHUNK = BPW // ICHUNK
L = 16
BK = 16384
GRID = (V + BK - 1) // BK


def _tc_matvec_body(w8_ref, tu_ref, tv_ref, qu_ref, qv_ref):
    qu_ref[...] = jnp.dot(w8_ref[0:8, :], tu_ref[...],
                          preferred_element_type=jnp.float32)[0]
    qv_ref[...] = jnp.dot(w8_ref[8:16, :], tv_ref[...],
                          preferred_element_type=jnp.float32)[0]


_tc_matvec = pl.pallas_call(
    _tc_matvec_body,
    grid=(GRID,),
    in_specs=[
        pl.BlockSpec((16, H), lambda i: (0, 0)),
        pl.BlockSpec((H, BK), lambda i: (0, i)),
        pl.BlockSpec((H, BK), lambda i: (0, i)),
    ],
    out_specs=[
        pl.BlockSpec((BK,), lambda i: (i,)),
        pl.BlockSpec((BK,), lambda i: (i,)),
    ],
    out_shape=[
        jax.ShapeDtypeStruct((V,), jnp.float32),
        jax.ShapeDtypeStruct((V,), jnp.float32),
    ],
    compiler_params=pltpu.CompilerParams(
        dimension_semantics=("arbitrary",)),
)


def _make_sc_kernel():
    mesh = plsc.VectorSubcoreMesh(core_axis_name="c", subcore_axis_name="s")

    @functools.partial(
        pl.kernel,
        mesh=mesh,
        out_type=jax.ShapeDtypeStruct((B,), jnp.float32),
        compiler_params=pltpu.CompilerParams(
            needs_layout_passes=False, use_tc_tiling_on_sc=False),
        scratch_types=[
            pltpu.VMEM((NCHUNK, ICHUNK), jnp.int32),
            pltpu.VMEM((NCHUNK, ICHUNK), jnp.int32),
            pltpu.VMEM((NCHUNK, ICHUNK), jnp.float32),
            pltpu.VMEM((NCHUNK, ICHUNK), jnp.float32),
            pltpu.VMEM((L,), jnp.float32),
            pltpu.VMEM((BPW,), jnp.float32),
            pltpu.SemaphoreType.DMA,
        ],
    )
    def k(u_hbm, v_hbm, cvec_hbm, qu_hbm, qv_hbm, out_hbm,
          idx_u, idx_v, val_u, val_v, cv, res, sem):
        wid = lax.axis_index("s") * 2 + lax.axis_index("c")
        base = wid * BPW

        pltpu.sync_copy(u_hbm.at[wid], idx_u)
        pltpu.sync_copy(v_hbm.at[wid], idx_v)
        pltpu.sync_copy(cvec_hbm, cv)

        copies = []
        for j in range(NCHUNK):
            copies.append(pltpu.async_copy(
                qu_hbm.at[idx_u.at[j]], val_u.at[j], sem))
            copies.append(pltpu.async_copy(
                qv_hbm.at[idx_v.at[j]], val_v.at[j], sem))
        for cp in copies:
            cp.wait()

        cvec = cv[...]
        for j in range(NCHUNK):
            for g in range(ICHUNK // L):
                a = val_u[j, pl.ds(g * L, L)]
                b = val_v[j, pl.ds(g * L, L)]
                z = 1.0 / (1.0 + jnp.exp(-(a + b + cvec)))
                res[pl.ds(j * ICHUNK + g * L, L)] = z

        pltpu.sync_copy(res, out_hbm.at[pl.ds(base, BPW)])

    return k


_sc_kernel = _make_sc_kernel()


def kernel(u, v, table_u, table_v, W1, gamma, beta, W2):
    scale = (gamma * jax.lax.rsqrt(jnp.float32(1.0 + 1e-5))) * W2[0]
    weff = scale @ W1
    c = jnp.dot(beta, W2[0])
    w8 = jnp.zeros((16, H), jnp.float32)
    w8 = w8.at[0].set(weff[:H]).at[8].set(weff[H:])
    cvec = jnp.full((L,), c, jnp.float32)

    qu, qv = _tc_matvec(w8, table_u.T, table_v.T)

    u3 = u.reshape(NW, NCHUNK, ICHUNK).astype(jnp.int32)
    v3 = v.reshape(NW, NCHUNK, ICHUNK).astype(jnp.int32)
    out = _sc_kernel(u3, v3, cvec, qu, qv)
    return out.reshape(B, 1)
